# baseline (speedup 1.0000x reference)
import functools

import jax
import jax.numpy as jnp
from jax import lax
from jax.experimental import pallas as pl
from jax.experimental.pallas import tpu as pltpu
from jax.experimental.pallas import tpu_sc as plsc

N = 10000
D = 128
HD = D // 2
G = 64
K = 128
RPT = 640
NP = 16 * RPT
NR = N // 16


def _cdiv(a, b):
    return (a + b - 1) // b


def _mesh():
    return plsc.VectorSubcoreMesh(core_axis_name="c", subcore_axis_name="s")


def _sc_degree(dst4, ones, zeros16):
    cpt = dst4.shape[2]

    @functools.partial(
        pl.kernel,
        out_type=jax.ShapeDtypeStruct((2, NP, 16), jnp.float32),
        mesh=_mesh(),
        compiler_params=pltpu.CompilerParams(use_tc_tiling_on_sc=False),
        scratch_types=[
            pltpu.VMEM((cpt, K), jnp.int32),
            pltpu.VMEM((K, 16), jnp.float32),
            pltpu.VMEM_SHARED((NP, 16), jnp.float32),
        ],
    )
    def k(dst_hbm, ones_hbm, z_hbm, out_hbm, dst_v, ones_v, acc_sh):
        c = lax.axis_index("c")
        s = lax.axis_index("s")
        r0 = s * RPT
        pltpu.sync_copy(z_hbm, acc_sh.at[pl.ds(r0, RPT)])
        pltpu.sync_copy(dst_hbm.at[c, s], dst_v)
        pltpu.sync_copy(ones_hbm, ones_v)
        plsc.subcore_barrier()

        def body(j, carry):
            pltpu.sync_copy(ones_v, acc_sh.at[dst_v.at[j]], add=True)
            return carry

        lax.fori_loop(0, cpt, body, 0)
        plsc.subcore_barrier()
        pltpu.sync_copy(acc_sh.at[pl.ds(r0, RPT)],
                        out_hbm.at[c, pl.ds(r0, RPT)])

    return k(dst4, ones, zeros16)


IB = 16


def _sc_scatter(ys, src4, dst4, zeros):
    cptd = src4.shape[2]
    cpt = 2 * cptd
    nbh = cptd // IB
    nb = 2 * nbh

    @functools.partial(
        pl.kernel,
        out_type=jax.ShapeDtypeStruct((2, NP, HD), jnp.float32),
        mesh=_mesh(),
        compiler_params=pltpu.CompilerParams(use_tc_tiling_on_sc=False),
        scratch_types=[
            pltpu.VMEM((2, IB, K), jnp.int32),
            pltpu.VMEM((2, IB, K), jnp.int32),
            [pltpu.VMEM((K, HD), jnp.float32)] * 3,
            pltpu.VMEM_SHARED((N, HD), jnp.float32),
            pltpu.VMEM_SHARED((NP, HD), jnp.float32),
            [pltpu.SemaphoreType.DMA] * 3,
            pltpu.SemaphoreType.DMA,
        ],
    )
    def k(y_hbm, src_hbm, dst_hbm, z_hbm, out_hbm,
          src_i, dst_i, rows, y_sh, acc_sh, sem_g, sem_i):
        c = lax.axis_index("c")
        s = lax.axis_index("s")
        r0 = s * RPT
        pltpu.sync_copy(z_hbm, acc_sh.at[pl.ds(r0, RPT)])
        pltpu.sync_copy(y_hbm.at[c].at[pl.ds(s * NR, NR)],
                        y_sh.at[pl.ds(s * NR, NR)])

        def iload(b, r, copy):
            h = b // nbh
            lb = b % nbh
            copy(src_hbm.at[h, s].at[pl.ds(lb * IB, IB)], src_i.at[r])
            copy(dst_hbm.at[h, s].at[pl.ds(lb * IB, IB)], dst_i.at[r])

        def iwait():
            pltpu.make_async_copy(src_hbm.at[0, s].at[pl.ds(0, IB)],
                                  src_i.at[0], sem_i).wait()
            pltpu.make_async_copy(dst_hbm.at[0, s].at[pl.ds(0, IB)],
                                  dst_i.at[0], sem_i).wait()

        def gwait(i):
            pltpu.make_async_copy(y_sh.at[src_i.at[0, 0]], rows[i],
                                  sem_g[i]).wait()

        iload(0, 0, pltpu.sync_copy)
        plsc.subcore_barrier()

        pltpu.async_copy(y_sh.at[src_i.at[0, 0]], rows[0], sem_g[0])

        def body(j, carry):
            b = j // IB
            jj = j - b * IB
            r = b % 2

            @pl.when((jj == 0) & (b + 1 < nb))
            def _iload():
                def acopy(a, d):
                    pltpu.async_copy(a, d, sem_i)
                iload(b + 1, (b + 1) % 2, acopy)

            for i in range(3):
                @pl.when(j % 3 == i)
                def _slot(i=i):
                    gwait(i)
                    pltpu.sync_copy(rows[i], acc_sh.at[dst_i.at[r, jj]],
                                    add=True)

                    @pl.when(j + 1 < cpt)
                    def _pref(i=i):
                        j1 = j + 1
                        b1 = j1 // IB
                        jj1 = j1 - b1 * IB
                        r1 = b1 % 2

                        @pl.when(jj1 == 0)
                        def _iw():
                            iwait()

                        @pl.when(j1 % 4 == 0)
                        def _gs():
                            pltpu.async_copy(y_sh.at[src_i.at[r1, jj1]],
                                             rows[(i + 1) % 3],
                                             sem_g[(i + 1) % 3])

                        @pl.when(j1 % 4 != 0)
                        def _gh():
                            pltpu.async_copy(y_hbm.at[c].at[src_i.at[r1, jj1]],
                                             rows[(i + 1) % 3],
                                             sem_g[(i + 1) % 3])

            return carry

        lax.fori_loop(0, cpt, body, 0)
        plsc.subcore_barrier()
        pltpu.sync_copy(acc_sh.at[pl.ds(r0, RPT)],
                        out_hbm.at[c, pl.ds(r0, RPT)])

    return k(ys, src4, dst4, zeros)


def _tc_first(x, W1, d0, d1):
    def body(x_ref, w_ref, d0_ref, d1_ref, y_ref, dinv_ref):
        deg = d0_ref[...] + d1_ref[...] + 1.0
        dinv = lax.rsqrt(jnp.maximum(deg, 1e-12))
        dinv_ref[...] = dinv
        y = jnp.dot(x_ref[...], w_ref[...],
                    preferred_element_type=jnp.float32) * dinv
        y_ref[0] = y[:, :HD]
        y_ref[1] = y[:, HD:]

    return pl.pallas_call(
        body,
        out_shape=(
            jax.ShapeDtypeStruct((2, N, HD), jnp.float32),
            jax.ShapeDtypeStruct((N, 1), jnp.float32),
        ),
    )(x, W1, d0, d1)


def _tc_mid(a0, a1, yprev, dinv, b, W):
    def body(a0_ref, a1_ref, yp_ref, di_ref, b_ref, w_ref, out_ref):
        dinv = di_ref[...]
        s = jnp.concatenate([a0_ref[...] + yp_ref[0],
                             a1_ref[...] + yp_ref[1]], axis=1)
        h = jnp.maximum(s * dinv + b_ref[...], 0.0)
        y = jnp.dot(h, w_ref[...],
                    preferred_element_type=jnp.float32) * dinv
        out_ref[0] = y[:, :HD]
        out_ref[1] = y[:, HD:]

    return pl.pallas_call(
        body,
        out_shape=jax.ShapeDtypeStruct((2, N, HD), jnp.float32),
    )(a0, a1, yprev, dinv, b, W)


def _tc_head(a0, a1, yprev, dinv, b3, batch_row, LW1, Lb1, LW2, Lb2):
    def body(a0_ref, a1_ref, yp_ref, di_ref, b_ref, batch_ref,
             lw1_ref, lb1_ref, lw2_ref, lb2_ref, out_ref):
        dinv = di_ref[...]
        s = jnp.concatenate([a0_ref[...] + yp_ref[0],
                             a1_ref[...] + yp_ref[1]], axis=1)
        h = jnp.maximum(s * dinv + b_ref[...], 0.0)
        gids = lax.broadcasted_iota(jnp.int32, (G, N), 0)
        onehot = (gids == batch_ref[...]).astype(jnp.float32)
        sums = jnp.dot(onehot, h, preferred_element_type=jnp.float32)
        cnts = jnp.sum(onehot, axis=1, keepdims=True)
        pooled = sums / jnp.maximum(cnts, 1.0)
        z = jnp.maximum(
            jnp.dot(pooled, lw1_ref[...],
                    preferred_element_type=jnp.float32) + lb1_ref[...], 0.0)
        logits = jnp.dot(z, lw2_ref[...],
                         preferred_element_type=jnp.float32) + lb2_ref[...]
        m = jnp.max(logits, axis=1, keepdims=True)
        sh = logits - m
        lse = jnp.log(jnp.sum(jnp.exp(sh), axis=1, keepdims=True))
        out_ref[...] = sh - lse

    return pl.pallas_call(
        body,
        out_shape=jax.ShapeDtypeStruct((G, 10), jnp.float32),
    )(a0, a1, yprev, dinv, b3, batch_row, LW1, Lb1, LW2, Lb2)


def kernel(x, edge_index, batch, W1, b1, W2, b2, W3, b3, LW1, Lb1, LW2, Lb2):
    E = edge_index.shape[1]
    eh = E // 2
    cptd = _cdiv(eh, 16 * K)
    cptd = _cdiv(cptd, IB) * IB
    epad = 16 * cptd * K
    pad = epad - eh

    def split(a, padval):
        padv = jnp.full((pad,), padval, jnp.int32)
        return jnp.concatenate(
            [a[:eh], padv, a[eh:], padv]).reshape(2, 16, cptd, K)

    src = split(edge_index[0], 0)
    dst = split(edge_index[1], NP - 1)

    ones16 = jnp.ones((K, 16), jnp.float32)
    zeros16 = jnp.zeros((RPT, 16), jnp.float32)
    zerosH = jnp.zeros((RPT, HD), jnp.float32)

    degp = _sc_degree(dst, ones16, zeros16)
    d0 = degp[0, :N, 0:1]
    d1 = degp[1, :N, 0:1]

    y1, dinv = _tc_first(x, W1, d0, d1)

    acc = _sc_scatter(y1, src, dst, zerosH)
    y2 = _tc_mid(acc[0, :N], acc[1, :N], y1, dinv, b1.reshape(1, D), W2)

    acc = _sc_scatter(y2, src, dst, zerosH)
    y3 = _tc_mid(acc[0, :N], acc[1, :N], y2, dinv, b2.reshape(1, D), W3)

    acc = _sc_scatter(y3, src, dst, zerosH)
    return _tc_head(acc[0, :N], acc[1, :N], y3, dinv, b3.reshape(1, D),
                    batch.reshape(1, N), LW1, Lb1.reshape(1, 64),
                    LW2, Lb2.reshape(1, 10))


# speedup vs baseline: 1.6484x; 1.6484x over previous
import functools

import jax
import jax.numpy as jnp
from jax import lax
from jax.experimental import pallas as pl
from jax.experimental.pallas import tpu as pltpu
from jax.experimental.pallas import tpu_sc as plsc

N = 10000
D = 128
HD = D // 2
G = 64
K = 128
RPT = 640
NP = 16 * RPT


def _cdiv(a, b):
    return (a + b - 1) // b


def _mesh():
    return plsc.VectorSubcoreMesh(core_axis_name="c", subcore_axis_name="s")


def _sc_degree(dst3, ones, zeros16):
    cpt = dst3.shape[1]
    hcpt = cpt // 2

    @functools.partial(
        pl.kernel,
        out_type=jax.ShapeDtypeStruct((2, NP, 16), jnp.float32),
        mesh=_mesh(),
        compiler_params=pltpu.CompilerParams(use_tc_tiling_on_sc=False),
        scratch_types=[
            pltpu.VMEM((cpt, K), jnp.int32),
            pltpu.VMEM((K, 16), jnp.float32),
            pltpu.VMEM_SHARED((NP, 16), jnp.float32),
        ],
    )
    def k(dst_hbm, ones_hbm, z_hbm, out_hbm, dst_v, ones_v, acc_sh):
        c = lax.axis_index("c")
        s = lax.axis_index("s")
        r0 = s * RPT
        pltpu.sync_copy(z_hbm, acc_sh.at[pl.ds(r0, RPT)])
        pltpu.sync_copy(dst_hbm.at[s], dst_v)
        pltpu.sync_copy(ones_hbm, ones_v)
        plsc.subcore_barrier()

        def body(j, carry):
            pltpu.sync_copy(ones_v, acc_sh.at[dst_v.at[j]], add=True)
            return carry

        lax.fori_loop(c * hcpt, (c + 1) * hcpt, body, 0)
        plsc.subcore_barrier()
        pltpu.sync_copy(acc_sh.at[pl.ds(r0, RPT)],
                        out_hbm.at[c, pl.ds(r0, RPT)])

    return k(dst3, ones, zeros16)


def _sc_scatter(ys, src3, dst3, zeros):
    cpt = src3.shape[1]

    @functools.partial(
        pl.kernel,
        out_type=jax.ShapeDtypeStruct((2, NP, HD), jnp.float32),
        mesh=_mesh(),
        compiler_params=pltpu.CompilerParams(use_tc_tiling_on_sc=False),
        scratch_types=[
            pltpu.VMEM((cpt, K), jnp.int32),
            pltpu.VMEM((cpt, K), jnp.int32),
            [pltpu.VMEM((K, HD), jnp.float32)] * 4,
            pltpu.VMEM_SHARED((NP, HD), jnp.float32),
            [pltpu.SemaphoreType.DMA] * 4,
            [pltpu.SemaphoreType.DMA] * 4,
        ],
    )
    def k(y_hbm, src_hbm, dst_hbm, z_hbm, out_hbm,
          src_v, dst_v, rows, acc_sh, sem_g, sem_s):
        c = lax.axis_index("c")
        s = lax.axis_index("s")
        r0 = s * RPT
        pltpu.sync_copy(z_hbm, acc_sh.at[pl.ds(r0, RPT)])
        pltpu.sync_copy(src_hbm.at[s], src_v)
        pltpu.sync_copy(dst_hbm.at[s], dst_v)
        plsc.subcore_barrier()

        yc = y_hbm.at[c]

        def gwait(i):
            pltpu.make_async_copy(yc.at[src_v.at[0]], rows[i],
                                  sem_g[i]).wait()

        def swait(i):
            pltpu.make_async_copy(rows[i], acc_sh.at[dst_v.at[0]],
                                  sem_s[i]).wait()

        pltpu.async_copy(yc.at[src_v.at[0]], rows[0], sem_g[0])
        pltpu.async_copy(yc.at[src_v.at[1]], rows[1], sem_g[1])

        def body(j, carry):
            for i in range(4):
                @pl.when(j % 4 == i)
                def _slot(i=i):
                    gwait(i)
                    pltpu.async_copy(rows[i], acc_sh.at[dst_v.at[j]],
                                     sem_s[i], add=True)
                    i2 = (i + 2) % 4

                    @pl.when(j + 2 < cpt)
                    def _pref():
                        @pl.when(j >= 2)
                        def _drain():
                            swait(i2)
                        pltpu.async_copy(yc.at[src_v.at[j + 2]], rows[i2],
                                         sem_g[i2])

            return carry

        lax.fori_loop(0, cpt, body, 0)
        for i in range(4):
            swait(i)
        plsc.subcore_barrier()
        pltpu.sync_copy(acc_sh.at[pl.ds(r0, RPT)],
                        out_hbm.at[c, pl.ds(r0, RPT)])

    return k(ys, src3, dst3, zeros)


def _tc_first(x, W1, d0, d1):
    def body(x_ref, w_ref, d0_ref, d1_ref, y_ref, dinv_ref):
        deg = d0_ref[...] + d1_ref[...] + 1.0
        dinv = lax.rsqrt(jnp.maximum(deg, 1e-12))
        dinv_ref[...] = dinv
        y = jnp.dot(x_ref[...], w_ref[...],
                    preferred_element_type=jnp.float32) * dinv
        y_ref[0] = y[:, :HD]
        y_ref[1] = y[:, HD:]

    return pl.pallas_call(
        body,
        out_shape=(
            jax.ShapeDtypeStruct((2, N, HD), jnp.float32),
            jax.ShapeDtypeStruct((N, 1), jnp.float32),
        ),
    )(x, W1, d0, d1)


def _tc_mid(a0, a1, yprev, dinv, b, W):
    def body(a0_ref, a1_ref, yp_ref, di_ref, b_ref, w_ref, out_ref):
        dinv = di_ref[...]
        s = jnp.concatenate([a0_ref[...] + yp_ref[0],
                             a1_ref[...] + yp_ref[1]], axis=1)
        h = jnp.maximum(s * dinv + b_ref[...], 0.0)
        y = jnp.dot(h, w_ref[...],
                    preferred_element_type=jnp.float32) * dinv
        out_ref[0] = y[:, :HD]
        out_ref[1] = y[:, HD:]

    return pl.pallas_call(
        body,
        out_shape=jax.ShapeDtypeStruct((2, N, HD), jnp.float32),
    )(a0, a1, yprev, dinv, b, W)


def _tc_head(a0, a1, yprev, dinv, b3, batch_row, LW1, Lb1, LW2, Lb2):
    def body(a0_ref, a1_ref, yp_ref, di_ref, b_ref, batch_ref,
             lw1_ref, lb1_ref, lw2_ref, lb2_ref, out_ref):
        dinv = di_ref[...]
        s = jnp.concatenate([a0_ref[...] + yp_ref[0],
                             a1_ref[...] + yp_ref[1]], axis=1)
        h = jnp.maximum(s * dinv + b_ref[...], 0.0)
        gids = lax.broadcasted_iota(jnp.int32, (G, N), 0)
        onehot = (gids == batch_ref[...]).astype(jnp.float32)
        sums = jnp.dot(onehot, h, preferred_element_type=jnp.float32)
        cnts = jnp.sum(onehot, axis=1, keepdims=True)
        pooled = sums / jnp.maximum(cnts, 1.0)
        z = jnp.maximum(
            jnp.dot(pooled, lw1_ref[...],
                    preferred_element_type=jnp.float32) + lb1_ref[...], 0.0)
        logits = jnp.dot(z, lw2_ref[...],
                         preferred_element_type=jnp.float32) + lb2_ref[...]
        m = jnp.max(logits, axis=1, keepdims=True)
        sh = logits - m
        lse = jnp.log(jnp.sum(jnp.exp(sh), axis=1, keepdims=True))
        out_ref[...] = sh - lse

    return pl.pallas_call(
        body,
        out_shape=jax.ShapeDtypeStruct((G, 10), jnp.float32),
    )(a0, a1, yprev, dinv, b3, batch_row, LW1, Lb1, LW2, Lb2)


def kernel(x, edge_index, batch, W1, b1, W2, b2, W3, b3, LW1, Lb1, LW2, Lb2):
    E = edge_index.shape[1]
    cpt = _cdiv(E, 16 * K)
    cpt = cpt + (cpt % 2)
    epad = 16 * cpt * K
    pad = epad - E

    src = jnp.concatenate(
        [edge_index[0], jnp.zeros((pad,), jnp.int32)]).reshape(16, cpt, K)
    dst = jnp.concatenate(
        [edge_index[1], jnp.full((pad,), NP - 1, jnp.int32)]).reshape(
            16, cpt, K)

    ones16 = jnp.ones((K, 16), jnp.float32)
    zeros16 = jnp.zeros((RPT, 16), jnp.float32)
    zerosH = jnp.zeros((RPT, HD), jnp.float32)

    degp = _sc_degree(dst, ones16, zeros16)
    d0 = degp[0, :N, 0:1]
    d1 = degp[1, :N, 0:1]

    y1, dinv = _tc_first(x, W1, d0, d1)

    acc = _sc_scatter(y1, src, dst, zerosH)
    y2 = _tc_mid(acc[0, :N], acc[1, :N], y1, dinv, b1.reshape(1, D), W2)

    acc = _sc_scatter(y2, src, dst, zerosH)
    y3 = _tc_mid(acc[0, :N], acc[1, :N], y2, dinv, b2.reshape(1, D), W3)

    acc = _sc_scatter(y3, src, dst, zerosH)
    return _tc_head(acc[0, :N], acc[1, :N], y3, dinv, b3.reshape(1, D),
                    batch.reshape(1, N), LW1, Lb1.reshape(1, 64),
                    LW2, Lb2.reshape(1, 10))


# speedup vs baseline: 1.7082x; 1.0363x over previous
import functools

import jax
import jax.numpy as jnp
from jax import lax
from jax.experimental import pallas as pl
from jax.experimental.pallas import tpu as pltpu
from jax.experimental.pallas import tpu_sc as plsc

N = 10000
D = 128
HD = D // 2
G = 64
K = 128
RPT = 640
NP = 16 * RPT


def _cdiv(a, b):
    return (a + b - 1) // b


def _mesh():
    return plsc.VectorSubcoreMesh(core_axis_name="c", subcore_axis_name="s")


def _sc_degree(dst3, ones, zeros16):
    cpt = dst3.shape[1]
    hcpt = cpt // 2

    @functools.partial(
        pl.kernel,
        out_type=jax.ShapeDtypeStruct((2, NP, 16), jnp.float32),
        mesh=_mesh(),
        compiler_params=pltpu.CompilerParams(use_tc_tiling_on_sc=False),
        scratch_types=[
            pltpu.VMEM((cpt, K), jnp.int32),
            pltpu.VMEM((K, 16), jnp.float32),
            pltpu.VMEM_SHARED((NP, 16), jnp.float32),
        ],
    )
    def k(dst_hbm, ones_hbm, z_hbm, out_hbm, dst_v, ones_v, acc_sh):
        c = lax.axis_index("c")
        s = lax.axis_index("s")
        r0 = s * RPT
        pltpu.sync_copy(z_hbm, acc_sh.at[pl.ds(r0, RPT)])
        pltpu.sync_copy(dst_hbm.at[s], dst_v)
        pltpu.sync_copy(ones_hbm, ones_v)
        plsc.subcore_barrier()

        def body(j, carry):
            pltpu.sync_copy(ones_v, acc_sh.at[dst_v.at[j]], add=True)
            return carry

        lax.fori_loop(c * hcpt, (c + 1) * hcpt, body, 0)
        plsc.subcore_barrier()
        pltpu.sync_copy(acc_sh.at[pl.ds(r0, RPT)],
                        out_hbm.at[c, pl.ds(r0, RPT)])

    return k(dst3, ones, zeros16)


def _sc_scatter(ys, src3, dst3, zeros):
    cpt = src3.shape[1]

    @functools.partial(
        pl.kernel,
        out_type=jax.ShapeDtypeStruct((2, NP, HD), jnp.float32),
        mesh=_mesh(),
        compiler_params=pltpu.CompilerParams(use_tc_tiling_on_sc=False),
        scratch_types=[
            pltpu.VMEM((cpt, K), jnp.int32),
            pltpu.VMEM((cpt, K), jnp.int32),
            [pltpu.VMEM((K, HD), jnp.float32)] * 6,
            pltpu.VMEM_SHARED((NP, HD), jnp.float32),
            [pltpu.SemaphoreType.DMA] * 6,
            [pltpu.SemaphoreType.DMA] * 6,
        ],
    )
    def k(y_hbm, src_hbm, dst_hbm, z_hbm, out_hbm,
          src_v, dst_v, rows, acc_sh, sem_g, sem_s):
        c = lax.axis_index("c")
        s = lax.axis_index("s")
        r0 = s * RPT
        pltpu.sync_copy(z_hbm, acc_sh.at[pl.ds(r0, RPT)])
        pltpu.sync_copy(src_hbm.at[s], src_v)
        pltpu.sync_copy(dst_hbm.at[s], dst_v)
        plsc.subcore_barrier()

        yc = y_hbm.at[c]

        def gwait(i):
            pltpu.make_async_copy(yc.at[src_v.at[0]], rows[i],
                                  sem_g[i]).wait()

        def swait(i):
            pltpu.make_async_copy(rows[i], acc_sh.at[dst_v.at[0]],
                                  sem_s[i]).wait()

        for i in range(3):
            pltpu.async_copy(yc.at[src_v.at[i]], rows[i], sem_g[i])

        def body(j, carry):
            for i in range(6):
                @pl.when(j % 6 == i)
                def _slot(i=i):
                    gwait(i)
                    pltpu.async_copy(rows[i], acc_sh.at[dst_v.at[j]],
                                     sem_s[i], add=True)
                    i2 = (i + 3) % 6

                    @pl.when(j + 3 < cpt)
                    def _pref():
                        @pl.when(j >= 3)
                        def _drain():
                            swait(i2)
                        pltpu.async_copy(yc.at[src_v.at[j + 3]], rows[i2],
                                         sem_g[i2])

            return carry

        lax.fori_loop(0, cpt, body, 0)
        for i in range(6):
            swait(i)
        plsc.subcore_barrier()
        pltpu.sync_copy(acc_sh.at[pl.ds(r0, RPT)],
                        out_hbm.at[c, pl.ds(r0, RPT)])

    return k(ys, src3, dst3, zeros)


def _tc_first(x, W1, d0, d1):
    def body(x_ref, w_ref, d0_ref, d1_ref, y_ref, dinv_ref):
        deg = d0_ref[...] + d1_ref[...] + 1.0
        dinv = lax.rsqrt(jnp.maximum(deg, 1e-12))
        dinv_ref[...] = dinv
        y = jnp.dot(x_ref[...], w_ref[...],
                    preferred_element_type=jnp.float32) * dinv
        y_ref[0] = y[:, :HD]
        y_ref[1] = y[:, HD:]

    return pl.pallas_call(
        body,
        out_shape=(
            jax.ShapeDtypeStruct((2, N, HD), jnp.float32),
            jax.ShapeDtypeStruct((N, 1), jnp.float32),
        ),
    )(x, W1, d0, d1)


def _tc_mid(a0, a1, yprev, dinv, b, W):
    def body(a0_ref, a1_ref, yp_ref, di_ref, b_ref, w_ref, out_ref):
        dinv = di_ref[...]
        s = jnp.concatenate([a0_ref[...] + yp_ref[0],
                             a1_ref[...] + yp_ref[1]], axis=1)
        h = jnp.maximum(s * dinv + b_ref[...], 0.0)
        y = jnp.dot(h, w_ref[...],
                    preferred_element_type=jnp.float32) * dinv
        out_ref[0] = y[:, :HD]
        out_ref[1] = y[:, HD:]

    return pl.pallas_call(
        body,
        out_shape=jax.ShapeDtypeStruct((2, N, HD), jnp.float32),
    )(a0, a1, yprev, dinv, b, W)


def _tc_head(a0, a1, yprev, dinv, b3, batch_row, LW1, Lb1, LW2, Lb2):
    def body(a0_ref, a1_ref, yp_ref, di_ref, b_ref, batch_ref,
             lw1_ref, lb1_ref, lw2_ref, lb2_ref, out_ref):
        dinv = di_ref[...]
        s = jnp.concatenate([a0_ref[...] + yp_ref[0],
                             a1_ref[...] + yp_ref[1]], axis=1)
        h = jnp.maximum(s * dinv + b_ref[...], 0.0)
        gids = lax.broadcasted_iota(jnp.int32, (G, N), 0)
        onehot = (gids == batch_ref[...]).astype(jnp.float32)
        sums = jnp.dot(onehot, h, preferred_element_type=jnp.float32)
        cnts = jnp.sum(onehot, axis=1, keepdims=True)
        pooled = sums / jnp.maximum(cnts, 1.0)
        z = jnp.maximum(
            jnp.dot(pooled, lw1_ref[...],
                    preferred_element_type=jnp.float32) + lb1_ref[...], 0.0)
        logits = jnp.dot(z, lw2_ref[...],
                         preferred_element_type=jnp.float32) + lb2_ref[...]
        m = jnp.max(logits, axis=1, keepdims=True)
        sh = logits - m
        lse = jnp.log(jnp.sum(jnp.exp(sh), axis=1, keepdims=True))
        out_ref[...] = sh - lse

    return pl.pallas_call(
        body,
        out_shape=jax.ShapeDtypeStruct((G, 10), jnp.float32),
    )(a0, a1, yprev, dinv, b3, batch_row, LW1, Lb1, LW2, Lb2)


def kernel(x, edge_index, batch, W1, b1, W2, b2, W3, b3, LW1, Lb1, LW2, Lb2):
    E = edge_index.shape[1]
    cpt = _cdiv(E, 16 * K)
    cpt = cpt + (cpt % 2)
    epad = 16 * cpt * K
    pad = epad - E

    src = jnp.concatenate(
        [edge_index[0], jnp.zeros((pad,), jnp.int32)]).reshape(16, cpt, K)
    dst = jnp.concatenate(
        [edge_index[1], jnp.full((pad,), NP - 1, jnp.int32)]).reshape(
            16, cpt, K)

    ones16 = jnp.ones((K, 16), jnp.float32)
    zeros16 = jnp.zeros((RPT, 16), jnp.float32)
    zerosH = jnp.zeros((RPT, HD), jnp.float32)

    degp = _sc_degree(dst, ones16, zeros16)
    d0 = degp[0, :N, 0:1]
    d1 = degp[1, :N, 0:1]

    y1, dinv = _tc_first(x, W1, d0, d1)

    acc = _sc_scatter(y1, src, dst, zerosH)
    y2 = _tc_mid(acc[0, :N], acc[1, :N], y1, dinv, b1.reshape(1, D), W2)

    acc = _sc_scatter(y2, src, dst, zerosH)
    y3 = _tc_mid(acc[0, :N], acc[1, :N], y2, dinv, b2.reshape(1, D), W3)

    acc = _sc_scatter(y3, src, dst, zerosH)
    return _tc_head(acc[0, :N], acc[1, :N], y3, dinv, b3.reshape(1, D),
                    batch.reshape(1, N), LW1, Lb1.reshape(1, 64),
                    LW2, Lb2.reshape(1, 10))


# speedup vs baseline: 1.8655x; 1.0921x over previous
import functools

import jax
import jax.numpy as jnp
from jax import lax
from jax.experimental import pallas as pl
from jax.experimental.pallas import tpu as pltpu
from jax.experimental.pallas import tpu_sc as plsc

N = 10000
D = 128
HD = D // 2
G = 64
K = 96
RPT = 640
NP = 16 * RPT


def _cdiv(a, b):
    return (a + b - 1) // b


def _mesh():
    return plsc.VectorSubcoreMesh(core_axis_name="c", subcore_axis_name="s")


def _sc_degree(dst3, ones, zeros16):
    cpt = dst3.shape[1]
    hcpt = cpt // 2

    @functools.partial(
        pl.kernel,
        out_type=jax.ShapeDtypeStruct((2, NP, 16), jnp.float32),
        mesh=_mesh(),
        compiler_params=pltpu.CompilerParams(use_tc_tiling_on_sc=False),
        scratch_types=[
            pltpu.VMEM((cpt, K), jnp.int32),
            pltpu.VMEM((K, 16), jnp.float32),
            pltpu.VMEM_SHARED((NP, 16), jnp.float32),
        ],
    )
    def k(dst_hbm, ones_hbm, z_hbm, out_hbm, dst_v, ones_v, acc_sh):
        c = lax.axis_index("c")
        s = lax.axis_index("s")
        r0 = s * RPT
        pltpu.sync_copy(z_hbm, acc_sh.at[pl.ds(r0, RPT)])
        pltpu.sync_copy(dst_hbm.at[s], dst_v)
        pltpu.sync_copy(ones_hbm, ones_v)
        plsc.subcore_barrier()

        def body(j, carry):
            pltpu.sync_copy(ones_v, acc_sh.at[dst_v.at[j]], add=True)
            return carry

        lax.fori_loop(c * hcpt, (c + 1) * hcpt, body, 0)
        plsc.subcore_barrier()
        pltpu.sync_copy(acc_sh.at[pl.ds(r0, RPT)],
                        out_hbm.at[c, pl.ds(r0, RPT)])

    return k(dst3, ones, zeros16)


def _sc_scatter(ys, src3, dst3, zeros):
    cpt = src3.shape[1]

    @functools.partial(
        pl.kernel,
        out_type=jax.ShapeDtypeStruct((2, NP, HD), jnp.float32),
        mesh=_mesh(),
        compiler_params=pltpu.CompilerParams(use_tc_tiling_on_sc=False),
        scratch_types=[
            pltpu.VMEM((cpt, K), jnp.int32),
            pltpu.VMEM((cpt, K), jnp.int32),
            [pltpu.VMEM((K, HD), jnp.float32)] * 7,
            pltpu.VMEM_SHARED((NP, HD), jnp.float32),
            [pltpu.SemaphoreType.DMA] * 7,
            [pltpu.SemaphoreType.DMA] * 7,
        ],
    )
    def k(y_hbm, src_hbm, dst_hbm, z_hbm, out_hbm,
          src_v, dst_v, rows, acc_sh, sem_g, sem_s):
        c = lax.axis_index("c")
        s = lax.axis_index("s")
        r0 = s * RPT
        pltpu.sync_copy(z_hbm, acc_sh.at[pl.ds(r0, RPT)])
        pltpu.sync_copy(src_hbm.at[s], src_v)
        pltpu.sync_copy(dst_hbm.at[s], dst_v)
        plsc.subcore_barrier()

        yc = y_hbm.at[c]

        def gwait(i):
            pltpu.make_async_copy(yc.at[src_v.at[0]], rows[i],
                                  sem_g[i]).wait()

        def swait(i):
            pltpu.make_async_copy(rows[i], acc_sh.at[dst_v.at[0]],
                                  sem_s[i]).wait()

        for i in range(3):
            pltpu.async_copy(yc.at[src_v.at[i]], rows[i], sem_g[i])

        def body(j, carry):
            for i in range(7):
                @pl.when(j % 7 == i)
                def _slot(i=i):
                    gwait(i)
                    pltpu.async_copy(rows[i], acc_sh.at[dst_v.at[j]],
                                     sem_s[i], add=True)
                    i2 = (i + 3) % 7

                    @pl.when(j + 3 < cpt)
                    def _pref():
                        @pl.when(j >= 4)
                        def _drain():
                            swait(i2)
                        pltpu.async_copy(yc.at[src_v.at[j + 3]], rows[i2],
                                         sem_g[i2])

            return carry

        lax.fori_loop(0, cpt, body, 0)
        for i in range(7):
            swait(i)
        plsc.subcore_barrier()
        pltpu.sync_copy(acc_sh.at[pl.ds(r0, RPT)],
                        out_hbm.at[c, pl.ds(r0, RPT)])

    return k(ys, src3, dst3, zeros)


def _tc_first(x, W1, d0, d1):
    def body(x_ref, w_ref, d0_ref, d1_ref, y_ref, dinv_ref):
        deg = d0_ref[...] + d1_ref[...] + 1.0
        dinv = lax.rsqrt(jnp.maximum(deg, 1e-12))
        dinv_ref[...] = dinv
        y = jnp.dot(x_ref[...], w_ref[...],
                    preferred_element_type=jnp.float32) * dinv
        y_ref[0] = y[:, :HD]
        y_ref[1] = y[:, HD:]

    return pl.pallas_call(
        body,
        out_shape=(
            jax.ShapeDtypeStruct((2, N, HD), jnp.float32),
            jax.ShapeDtypeStruct((N, 1), jnp.float32),
        ),
    )(x, W1, d0, d1)


def _tc_mid(a0, a1, yprev, dinv, b, W):
    def body(a0_ref, a1_ref, yp_ref, di_ref, b_ref, w_ref, out_ref):
        dinv = di_ref[...]
        s = jnp.concatenate([a0_ref[...] + yp_ref[0],
                             a1_ref[...] + yp_ref[1]], axis=1)
        h = jnp.maximum(s * dinv + b_ref[...], 0.0)
        y = jnp.dot(h, w_ref[...],
                    preferred_element_type=jnp.float32) * dinv
        out_ref[0] = y[:, :HD]
        out_ref[1] = y[:, HD:]

    return pl.pallas_call(
        body,
        out_shape=jax.ShapeDtypeStruct((2, N, HD), jnp.float32),
    )(a0, a1, yprev, dinv, b, W)


def _tc_head(a0, a1, yprev, dinv, b3, batch_row, LW1, Lb1, LW2, Lb2):
    def body(a0_ref, a1_ref, yp_ref, di_ref, b_ref, batch_ref,
             lw1_ref, lb1_ref, lw2_ref, lb2_ref, out_ref):
        dinv = di_ref[...]
        s = jnp.concatenate([a0_ref[...] + yp_ref[0],
                             a1_ref[...] + yp_ref[1]], axis=1)
        h = jnp.maximum(s * dinv + b_ref[...], 0.0)
        gids = lax.broadcasted_iota(jnp.int32, (G, N), 0)
        onehot = (gids == batch_ref[...]).astype(jnp.float32)
        sums = jnp.dot(onehot, h, preferred_element_type=jnp.float32)
        cnts = jnp.sum(onehot, axis=1, keepdims=True)
        pooled = sums / jnp.maximum(cnts, 1.0)
        z = jnp.maximum(
            jnp.dot(pooled, lw1_ref[...],
                    preferred_element_type=jnp.float32) + lb1_ref[...], 0.0)
        logits = jnp.dot(z, lw2_ref[...],
                         preferred_element_type=jnp.float32) + lb2_ref[...]
        m = jnp.max(logits, axis=1, keepdims=True)
        sh = logits - m
        lse = jnp.log(jnp.sum(jnp.exp(sh), axis=1, keepdims=True))
        out_ref[...] = sh - lse

    return pl.pallas_call(
        body,
        out_shape=jax.ShapeDtypeStruct((G, 10), jnp.float32),
    )(a0, a1, yprev, dinv, b3, batch_row, LW1, Lb1, LW2, Lb2)


def kernel(x, edge_index, batch, W1, b1, W2, b2, W3, b3, LW1, Lb1, LW2, Lb2):
    E = edge_index.shape[1]
    cpt = _cdiv(E, 16 * K)
    cpt = cpt + (cpt % 2)
    epad = 16 * cpt * K
    pad = epad - E

    src = jnp.concatenate(
        [edge_index[0], jnp.zeros((pad,), jnp.int32)]).reshape(16, cpt, K)
    dst = jnp.concatenate(
        [edge_index[1], jnp.full((pad,), NP - 1, jnp.int32)]).reshape(
            16, cpt, K)

    ones16 = jnp.ones((K, 16), jnp.float32)
    zeros16 = jnp.zeros((RPT, 16), jnp.float32)
    zerosH = jnp.zeros((RPT, HD), jnp.float32)

    degp = _sc_degree(dst, ones16, zeros16)
    d0 = degp[0, :N, 0:1]
    d1 = degp[1, :N, 0:1]

    y1, dinv = _tc_first(x, W1, d0, d1)

    acc = _sc_scatter(y1, src, dst, zerosH)
    y2 = _tc_mid(acc[0, :N], acc[1, :N], y1, dinv, b1.reshape(1, D), W2)

    acc = _sc_scatter(y2, src, dst, zerosH)
    y3 = _tc_mid(acc[0, :N], acc[1, :N], y2, dinv, b2.reshape(1, D), W3)

    acc = _sc_scatter(y3, src, dst, zerosH)
    return _tc_head(acc[0, :N], acc[1, :N], y3, dinv, b3.reshape(1, D),
                    batch.reshape(1, N), LW1, Lb1.reshape(1, 64),
                    LW2, Lb2.reshape(1, 10))


# speedup vs baseline: 2.1422x; 1.1483x over previous
import functools

import jax
import jax.numpy as jnp
from jax import lax
from jax.experimental import pallas as pl
from jax.experimental.pallas import tpu as pltpu
from jax.experimental.pallas import tpu_sc as plsc

N = 10000
D = 128
HD = D // 2
G = 64
K = 64
RPT = 640
NP = 16 * RPT


def _cdiv(a, b):
    return (a + b - 1) // b


def _mesh():
    return plsc.VectorSubcoreMesh(core_axis_name="c", subcore_axis_name="s")


def _sc_degree(dst3, ones, zeros16):
    cpt = dst3.shape[1]
    hcpt = cpt // 2

    @functools.partial(
        pl.kernel,
        out_type=jax.ShapeDtypeStruct((2, NP, 16), jnp.float32),
        mesh=_mesh(),
        compiler_params=pltpu.CompilerParams(use_tc_tiling_on_sc=False),
        scratch_types=[
            pltpu.VMEM((cpt, K), jnp.int32),
            pltpu.VMEM((K, 16), jnp.float32),
            pltpu.VMEM_SHARED((NP, 16), jnp.float32),
        ],
    )
    def k(dst_hbm, ones_hbm, z_hbm, out_hbm, dst_v, ones_v, acc_sh):
        c = lax.axis_index("c")
        s = lax.axis_index("s")
        r0 = s * RPT
        pltpu.sync_copy(z_hbm, acc_sh.at[pl.ds(r0, RPT)])
        pltpu.sync_copy(dst_hbm.at[s], dst_v)
        pltpu.sync_copy(ones_hbm, ones_v)
        plsc.subcore_barrier()

        def body(j, carry):
            pltpu.sync_copy(ones_v, acc_sh.at[dst_v.at[j]], add=True)
            return carry

        lax.fori_loop(c * hcpt, (c + 1) * hcpt, body, 0)
        plsc.subcore_barrier()
        pltpu.sync_copy(acc_sh.at[pl.ds(r0, RPT)],
                        out_hbm.at[c, pl.ds(r0, RPT)])

    return k(dst3, ones, zeros16)


def _sc_scatter(ys, src3, dst3, zeros):
    cpt = src3.shape[1]

    @functools.partial(
        pl.kernel,
        out_type=jax.ShapeDtypeStruct((2, NP, HD), jnp.float32),
        mesh=_mesh(),
        compiler_params=pltpu.CompilerParams(use_tc_tiling_on_sc=False),
        scratch_types=[
            pltpu.VMEM((cpt, K), jnp.int32),
            pltpu.VMEM((cpt, K), jnp.int32),
            [pltpu.VMEM((K, HD), jnp.float32)] * 10,
            pltpu.VMEM_SHARED((NP, HD), jnp.float32),
            [pltpu.SemaphoreType.DMA] * 10,
            [pltpu.SemaphoreType.DMA] * 10,
        ],
    )
    def k(y_hbm, src_hbm, dst_hbm, z_hbm, out_hbm,
          src_v, dst_v, rows, acc_sh, sem_g, sem_s):
        c = lax.axis_index("c")
        s = lax.axis_index("s")
        r0 = s * RPT
        pltpu.sync_copy(z_hbm, acc_sh.at[pl.ds(r0, RPT)])
        pltpu.sync_copy(src_hbm.at[s], src_v)
        pltpu.sync_copy(dst_hbm.at[s], dst_v)
        plsc.subcore_barrier()

        yc = y_hbm.at[c]

        def gwait(i):
            pltpu.make_async_copy(yc.at[src_v.at[0]], rows[i],
                                  sem_g[i]).wait()

        def swait(i):
            pltpu.make_async_copy(rows[i], acc_sh.at[dst_v.at[0]],
                                  sem_s[i]).wait()

        for i in range(5):
            pltpu.async_copy(yc.at[src_v.at[i]], rows[i], sem_g[i])

        def body(j, carry):
            for i in range(10):
                @pl.when(j % 10 == i)
                def _slot(i=i):
                    gwait(i)
                    pltpu.async_copy(rows[i], acc_sh.at[dst_v.at[j]],
                                     sem_s[i], add=True)
                    i2 = (i + 5) % 10

                    @pl.when(j + 5 < cpt)
                    def _pref():
                        @pl.when(j >= 5)
                        def _drain():
                            swait(i2)
                        pltpu.async_copy(yc.at[src_v.at[j + 5]], rows[i2],
                                         sem_g[i2])

            return carry

        lax.fori_loop(0, cpt, body, 0)
        for i in range(10):
            swait(i)
        plsc.subcore_barrier()
        pltpu.sync_copy(acc_sh.at[pl.ds(r0, RPT)],
                        out_hbm.at[c, pl.ds(r0, RPT)])

    return k(ys, src3, dst3, zeros)


def _tc_first(x, W1, d0, d1):
    def body(x_ref, w_ref, d0_ref, d1_ref, y_ref, dinv_ref):
        deg = d0_ref[...] + d1_ref[...] + 1.0
        dinv = lax.rsqrt(jnp.maximum(deg, 1e-12))
        dinv_ref[...] = dinv
        y = jnp.dot(x_ref[...], w_ref[...],
                    preferred_element_type=jnp.float32) * dinv
        y_ref[0] = y[:, :HD]
        y_ref[1] = y[:, HD:]

    return pl.pallas_call(
        body,
        out_shape=(
            jax.ShapeDtypeStruct((2, N, HD), jnp.float32),
            jax.ShapeDtypeStruct((N, 1), jnp.float32),
        ),
    )(x, W1, d0, d1)


def _tc_mid(a0, a1, yprev, dinv, b, W):
    def body(a0_ref, a1_ref, yp_ref, di_ref, b_ref, w_ref, out_ref):
        dinv = di_ref[...]
        s = jnp.concatenate([a0_ref[...] + yp_ref[0],
                             a1_ref[...] + yp_ref[1]], axis=1)
        h = jnp.maximum(s * dinv + b_ref[...], 0.0)
        y = jnp.dot(h, w_ref[...],
                    preferred_element_type=jnp.float32) * dinv
        out_ref[0] = y[:, :HD]
        out_ref[1] = y[:, HD:]

    return pl.pallas_call(
        body,
        out_shape=jax.ShapeDtypeStruct((2, N, HD), jnp.float32),
    )(a0, a1, yprev, dinv, b, W)


def _tc_head(a0, a1, yprev, dinv, b3, batch_row, LW1, Lb1, LW2, Lb2):
    def body(a0_ref, a1_ref, yp_ref, di_ref, b_ref, batch_ref,
             lw1_ref, lb1_ref, lw2_ref, lb2_ref, out_ref):
        dinv = di_ref[...]
        s = jnp.concatenate([a0_ref[...] + yp_ref[0],
                             a1_ref[...] + yp_ref[1]], axis=1)
        h = jnp.maximum(s * dinv + b_ref[...], 0.0)
        gids = lax.broadcasted_iota(jnp.int32, (G, N), 0)
        onehot = (gids == batch_ref[...]).astype(jnp.float32)
        sums = jnp.dot(onehot, h, preferred_element_type=jnp.float32)
        cnts = jnp.sum(onehot, axis=1, keepdims=True)
        pooled = sums / jnp.maximum(cnts, 1.0)
        z = jnp.maximum(
            jnp.dot(pooled, lw1_ref[...],
                    preferred_element_type=jnp.float32) + lb1_ref[...], 0.0)
        logits = jnp.dot(z, lw2_ref[...],
                         preferred_element_type=jnp.float32) + lb2_ref[...]
        m = jnp.max(logits, axis=1, keepdims=True)
        sh = logits - m
        lse = jnp.log(jnp.sum(jnp.exp(sh), axis=1, keepdims=True))
        out_ref[...] = sh - lse

    return pl.pallas_call(
        body,
        out_shape=jax.ShapeDtypeStruct((G, 10), jnp.float32),
    )(a0, a1, yprev, dinv, b3, batch_row, LW1, Lb1, LW2, Lb2)


def kernel(x, edge_index, batch, W1, b1, W2, b2, W3, b3, LW1, Lb1, LW2, Lb2):
    E = edge_index.shape[1]
    cpt = _cdiv(E, 16 * K)
    cpt = cpt + (cpt % 2)
    epad = 16 * cpt * K
    pad = epad - E

    src = jnp.concatenate(
        [edge_index[0], jnp.zeros((pad,), jnp.int32)]).reshape(16, cpt, K)
    dst = jnp.concatenate(
        [edge_index[1], jnp.full((pad,), NP - 1, jnp.int32)]).reshape(
            16, cpt, K)

    ones16 = jnp.ones((K, 16), jnp.float32)
    zeros16 = jnp.zeros((RPT, 16), jnp.float32)
    zerosH = jnp.zeros((RPT, HD), jnp.float32)

    degp = _sc_degree(dst, ones16, zeros16)
    d0 = degp[0, :N, 0:1]
    d1 = degp[1, :N, 0:1]

    y1, dinv = _tc_first(x, W1, d0, d1)

    acc = _sc_scatter(y1, src, dst, zerosH)
    y2 = _tc_mid(acc[0, :N], acc[1, :N], y1, dinv, b1.reshape(1, D), W2)

    acc = _sc_scatter(y2, src, dst, zerosH)
    y3 = _tc_mid(acc[0, :N], acc[1, :N], y2, dinv, b2.reshape(1, D), W3)

    acc = _sc_scatter(y3, src, dst, zerosH)
    return _tc_head(acc[0, :N], acc[1, :N], y3, dinv, b3.reshape(1, D),
                    batch.reshape(1, N), LW1, Lb1.reshape(1, 64),
                    LW2, Lb2.reshape(1, 10))
